# baseline (speedup 1.0000x reference)
import functools

import jax
import jax.numpy as jnp
from jax import lax
from jax.experimental import pallas as pl
from jax.experimental.pallas import tpu as pltpu
from jax.experimental.pallas import tpu_sc as plsc

B, E, N, H = 4, 16384, 384, 32
NP1 = N + 1
PLANE = NP1 * NP1
HG = 4
REG = 8 * PLANE
NT = 16
EPT = E // NT
HALF0 = REG // 2 // 8 * 8
HALF1 = REG - HALF0
CH = HALF0 // NT
TAIL = HALF1 - NT * CH
RT = 48
NRT = NP1 // RT + 1


def _edge_body(ef_ref, ei_ref, nl_ref, em_ref, st_ref, pp_ref, pg_ref, pi_ref,
               dw1_ref, db1_ref, dw2_ref, db2_ref, emb_ref, idx_ref):
    f32 = jnp.float32
    e0 = ef_ref[0, 0:1, :]
    e1 = ef_ref[0, 1:2, :]
    e2 = ef_ref[0, 2:3, :]
    d = ef_ref[0, 3:4, :]
    t0 = e0.astype(jnp.int32)
    t1 = e1.astype(jnp.int32)
    t2 = e2.astype(jnp.int32)
    src = ei_ref[0, 0:1, :]
    tgt = ei_ref[0, 1:2, :]
    nl = jnp.maximum(nl_ref[0, 0, 0], 1)
    src_lig = (src > 0) & (src < nl)
    tgt_lig = (tgt > 0) & (tgt < nl)

    h = jnp.maximum(dw1_ref[...] * d + db1_ref[0][:, None], 0.0)
    dist_t = lax.dot_general(dw2_ref[...], h, (((1,), (0,)), ((), ())),
                             preferred_element_type=f32) + db2_ref[0][:, None]

    sidx = jnp.clip(t0 * 4 + t1 * 2 + t2, 0, 19)
    oh20 = (lax.broadcasted_iota(jnp.int32, (20, E), 0) == sidx).astype(f32)
    struct_t = lax.dot_general(st_ref[...], oh20, (((0,), (0,)), ((), ())),
                               preferred_element_type=f32)

    pidx = jnp.clip(t1, 0, 14)
    oh15 = (lax.broadcasted_iota(jnp.int32, (15, E), 0) == pidx).astype(f32)
    pro_t = lax.dot_general(pp_ref[...], oh15, (((0,), (0,)), ((), ())),
                            preferred_element_type=f32)
    lig_t = lax.dot_general(pg_ref[...], oh15, (((0,), (0,)), ((), ())),
                            preferred_element_type=f32)
    inter_t = lax.dot_general(pi_ref[...], oh15, (((0,), (0,)), ((), ())),
                              preferred_element_type=f32)

    both_lig = src_lig & tgt_lig
    both_pro = (~src_lig) & (~tgt_lig)
    plip_t = jnp.where(both_lig, lig_t, jnp.where(both_pro, pro_t, inter_t))

    is_struct = t0 <= 1
    is_plip = t0 == 5
    emb = jnp.where(is_struct, struct_t,
                    jnp.where(is_plip, plip_t, 0.0)) + dist_t

    inb = ((src >= 0) & (src < N) & (tgt >= 0) & (tgt < N)).astype(f32)
    emb = emb * (em_ref[0, 0:1, :] * inb)

    off = (jnp.clip(src, 0, N - 1) * NP1 + jnp.clip(tgt, 0, N - 1)
           + NP1 + 1)
    idx8 = off + lax.broadcasted_iota(jnp.int32, (8, E), 0) * PLANE
    inr0 = idx8 < HALF0
    for cc in range(2):
        inr = inr0 if cc == 0 else ~inr0
        idx_c = jnp.clip(idx8 - cc * HALF0, 0,
                         (HALF0 if cc == 0 else HALF1) - 1)
        z = idx_c.reshape(8, NT, 8, 128)
        idx_ref[0, cc] = jnp.transpose(z, (1, 0, 2, 3)).reshape(NT, 64, 128)
        vc = (emb.reshape(HG, 8, E) * inr.astype(f32)).reshape(HG, 8, NT, 8,
                                                               128)
        emb_ref[0, cc] = jnp.transpose(vc, (0, 2, 1, 3, 4)).reshape(
            HG, NT, 64, 128)


def _dense_body(ang_ref, dst_ref, w1_ref, b1_ref, w2_ref, b2_ref, g_ref,
                out_ref):
    m = pl.program_id(1)
    f32 = jnp.float32
    x = jnp.concatenate([ang_ref[0], dst_ref[0]], axis=-1).reshape(RT * N, 56)
    h = jnp.maximum(
        lax.dot_general(w1_ref[...], x, (((0,), (1,)), ((), ())),
                        preferred_element_type=f32) + b1_ref[0][:, None], 0.0)
    y = lax.dot_general(w2_ref[...], h, (((0,), (0,)), ((), ())),
                        preferred_element_type=f32) + b2_ref[0][:, None]
    y = y.reshape(H, RT, N)
    g = g_ref[0]
    gcol = jnp.broadcast_to(g[:, None, None], (H, RT, 1))
    out_ref[0] = jnp.concatenate([gcol, y], axis=2)

    @pl.when(m == 0)
    def _():
        out_ref[0, :, 0:1, :] = jnp.broadcast_to(g[:, None, None],
                                                 (H, 1, NP1))


def _sc_body(dense_hbm, emb_hbm, idx_hbm, out_hbm, idx_v, val_v, spmem,
             vbuf, tbuf, sem):
    c = lax.axis_index("c")
    s = lax.axis_index("s")
    start_c = c * HALF0
    for b in range(B):
        pltpu.sync_copy(idx_hbm.at[b, pl.ds(c, 1), pl.ds(s, 1)], idx_v)
        for g in range(HG):
            base = b * (HG * REG) + g * REG
            off = base + start_c + s * CH
            pltpu.sync_copy(dense_hbm.at[pl.ds(off, CH)], vbuf)
            pltpu.sync_copy(vbuf, spmem.at[pl.ds(s * CH, CH)])

            @pl.when((s == NT - 1) & (c == 1))
            def _():
                pltpu.sync_copy(
                    dense_hbm.at[pl.ds(base + HALF0 + NT * CH, TAIL)], tbuf)
                pltpu.sync_copy(tbuf, spmem.at[pl.ds(NT * CH, TAIL)])

            pltpu.sync_copy(
                emb_hbm.at[b, pl.ds(c, 1), g, pl.ds(s, 1)], val_v)
            plsc.subcore_barrier()
            cps = [
                pltpu.async_copy(val_v.at[0, 0, q],
                                 spmem.at[idx_v.at[0, 0, q]], sem, add=True)
                for q in range(64)
            ]
            for cp in cps:
                cp.wait()
            plsc.subcore_barrier()
            pltpu.sync_copy(spmem.at[pl.ds(s * CH, CH)], vbuf)
            pltpu.sync_copy(vbuf, out_hbm.at[pl.ds(off, CH)])

            @pl.when((s == NT - 1) & (c == 1))
            def _():
                pltpu.sync_copy(spmem.at[pl.ds(NT * CH, TAIL)], tbuf)
                pltpu.sync_copy(
                    tbuf, out_hbm.at[pl.ds(base + HALF0 + NT * CH, TAIL)])

            plsc.subcore_barrier()


def _edge_call(ef_t, ei, nl2, em_f, struct_tab, plip_pro, plip_lig, plip_inter,
               dist_w1, dist_b1, dist_w2, dist_b2):
    f32 = jnp.float32
    full = lambda shp: pl.BlockSpec(shp, lambda b: (0,) * len(shp))
    return pl.pallas_call(
        _edge_body,
        grid=(B,),
        in_specs=[
            pl.BlockSpec((1, 4, E), lambda b: (b, 0, 0)),
            pl.BlockSpec((1, 2, E), lambda b: (b, 0, 0)),
            pl.BlockSpec((1, 1, 1), lambda b: (b, 0, 0)),
            pl.BlockSpec((1, 1, E), lambda b: (b, 0, 0)),
            full((20, H)), full((15, H)), full((15, H)), full((15, H)),
            full((H, 1)), full((1, H)), full((H, H)), full((1, H)),
        ],
        out_specs=[
            pl.BlockSpec((1, 2, HG, NT, 64, 128),
                         lambda b: (b, 0, 0, 0, 0, 0)),
            pl.BlockSpec((1, 2, NT, 64, 128), lambda b: (b, 0, 0, 0, 0)),
        ],
        out_shape=[
            jax.ShapeDtypeStruct((B, 2, HG, NT, 64, 128), f32),
            jax.ShapeDtypeStruct((B, 2, NT, 64, 128), jnp.int32),
        ],
    )(ef_t, ei, nl2, em_f, struct_tab, plip_pro, plip_lig, plip_inter,
      dist_w1, dist_b1, dist_w2, dist_b2)


def _dense_call(angle_p, dists_p, w1c, b1c, w2c, b2c, gtvd):
    f32 = jnp.float32
    full = lambda shp: pl.BlockSpec(shp, lambda b, m: (0,) * len(shp))
    return pl.pallas_call(
        _dense_body,
        grid=(B, NRT),
        in_specs=[
            pl.BlockSpec((1, RT, N, 28), lambda b, m: (b, m, 0, 0)),
            pl.BlockSpec((1, RT, N, 28), lambda b, m: (b, m, 0, 0)),
            full((56, 64)), full((1, 64)), full((64, H)), full((1, H)),
            full((1, H)),
        ],
        out_specs=pl.BlockSpec((1, H, RT, NP1), lambda b, m: (b, 0, m, 0)),
        out_shape=jax.ShapeDtypeStruct((B, H, NP1, NP1), f32),
    )(angle_p, dists_p, w1c, b1c, w2c, b2c, gtvd)


@functools.cache
def _sc_scatter_fn():
    return pl.kernel(
        _sc_body,
        out_type=jax.ShapeDtypeStruct((B * HG * REG,), jnp.float32),
        mesh=plsc.VectorSubcoreMesh(
            core_axis_name="c", subcore_axis_name="s", num_cores=2),
        scratch_types=[
            pltpu.VMEM((1, 1, 64, 128), jnp.int32),
            pltpu.VMEM((1, 1, 64, 128), jnp.float32),
            pltpu.VMEM_SHARED((HALF1,), jnp.float32),
            pltpu.VMEM((CH,), jnp.float32),
            pltpu.VMEM((TAIL,), jnp.float32),
            pltpu.SemaphoreType.DMA,
        ],
    )


def kernel(edge_feat, edge_index, node_feat, num_ligand_atoms, angle, dists,
           edge_mask, struct_tab, plip_pro, plip_lig, plip_inter,
           dist_w1, dist_b1, dist_w2, dist_b2, gtvd,
           ang_w1, ang_b1, ang_w2, ang_b2, md_w1, md_b1, md_w2, md_b2):
    f32 = jnp.float32
    ef_t = jnp.transpose(edge_feat.astype(f32), (0, 2, 1))
    ei = edge_index.astype(jnp.int32)
    nl2 = num_ligand_atoms.astype(jnp.int32).reshape(B, 1, 1)
    em_f = edge_mask.astype(f32).reshape(B, 1, E)

    emb7, idx7 = _edge_call(
        ef_t, ei, nl2, em_f, struct_tab, plip_pro, plip_lig, plip_inter,
        dist_w1, dist_b1.reshape(1, H), dist_w2, dist_b2.reshape(1, H))

    w1c = jnp.zeros((56, 64), f32)
    w1c = w1c.at[:28, :32].set(ang_w1.T).at[28:, 32:].set(md_w1.T)
    b1c = jnp.concatenate([ang_b1, md_b1]).reshape(1, 64)
    w2c = jnp.concatenate([ang_w2.T, md_w2.T], axis=0)
    b2c = (ang_b2 + md_b2).reshape(1, H)

    pad = [(0, 0), (1, 0), (0, 0), (0, 0)]
    dense = _dense_call(jnp.pad(angle, pad), jnp.pad(dists, pad),
                        w1c, b1c, w2c, b2c, gtvd)

    out = _sc_scatter_fn()(dense.reshape(B * HG * REG), emb7, idx7)
    return out.reshape(B, H, NP1, NP1)


# speedup vs baseline: 1.0012x; 1.0012x over previous
import functools

import jax
import jax.numpy as jnp
from jax import lax
from jax.experimental import pallas as pl
from jax.experimental.pallas import tpu as pltpu
from jax.experimental.pallas import tpu_sc as plsc

B, E, N, H = 4, 16384, 384, 32
NP1 = N + 1
PLANE = NP1 * NP1
HG = 4
REG = 8 * PLANE
NT = 16
EPT = E // NT
HALF0 = REG // 2 // 8 * 8
HALF1 = REG - HALF0
CH = HALF0 // NT
TAIL = HALF1 - NT * CH
RT = 48
NRT = NP1 // RT + 1


def _edge_body(ef_ref, ei_ref, nl_ref, em_ref, st_ref, pp_ref, pg_ref, pi_ref,
               dw1_ref, db1_ref, dw2_ref, db2_ref, emb_ref, idx_ref):
    f32 = jnp.float32
    e0 = ef_ref[0, 0:1, :]
    e1 = ef_ref[0, 1:2, :]
    e2 = ef_ref[0, 2:3, :]
    d = ef_ref[0, 3:4, :]
    t0 = e0.astype(jnp.int32)
    t1 = e1.astype(jnp.int32)
    t2 = e2.astype(jnp.int32)
    src = ei_ref[0, 0:1, :]
    tgt = ei_ref[0, 1:2, :]
    nl = jnp.maximum(nl_ref[0, 0, 0], 1)
    src_lig = (src > 0) & (src < nl)
    tgt_lig = (tgt > 0) & (tgt < nl)

    h = jnp.maximum(dw1_ref[...] * d + db1_ref[0][:, None], 0.0)
    dist_t = lax.dot_general(dw2_ref[...], h, (((1,), (0,)), ((), ())),
                             preferred_element_type=f32) + db2_ref[0][:, None]

    sidx = jnp.clip(t0 * 4 + t1 * 2 + t2, 0, 19)
    oh20 = (lax.broadcasted_iota(jnp.int32, (20, E), 0) == sidx).astype(f32)
    struct_t = lax.dot_general(st_ref[...], oh20, (((0,), (0,)), ((), ())),
                               preferred_element_type=f32)

    pidx = jnp.clip(t1, 0, 14)
    oh15 = (lax.broadcasted_iota(jnp.int32, (15, E), 0) == pidx).astype(f32)
    pro_t = lax.dot_general(pp_ref[...], oh15, (((0,), (0,)), ((), ())),
                            preferred_element_type=f32)
    lig_t = lax.dot_general(pg_ref[...], oh15, (((0,), (0,)), ((), ())),
                            preferred_element_type=f32)
    inter_t = lax.dot_general(pi_ref[...], oh15, (((0,), (0,)), ((), ())),
                              preferred_element_type=f32)

    both_lig = src_lig & tgt_lig
    both_pro = (~src_lig) & (~tgt_lig)
    plip_t = jnp.where(both_lig, lig_t, jnp.where(both_pro, pro_t, inter_t))

    is_struct = t0 <= 1
    is_plip = t0 == 5
    emb = jnp.where(is_struct, struct_t,
                    jnp.where(is_plip, plip_t, 0.0)) + dist_t

    inb = ((src >= 0) & (src < N) & (tgt >= 0) & (tgt < N)).astype(f32)
    emb = emb * (em_ref[0, 0:1, :] * inb)

    off = (jnp.clip(src, 0, N - 1) * NP1 + jnp.clip(tgt, 0, N - 1)
           + NP1 + 1)
    idx8 = off + lax.broadcasted_iota(jnp.int32, (8, E), 0) * PLANE
    inr0 = idx8 < HALF0
    for cc in range(2):
        inr = inr0 if cc == 0 else ~inr0
        idx_c = jnp.clip(idx8 - cc * HALF0, 0,
                         (HALF0 if cc == 0 else HALF1) - 1)
        z = idx_c.reshape(8, NT, 8, 128)
        idx_ref[0, cc] = jnp.transpose(z, (1, 0, 2, 3)).reshape(NT, 8192)
        vc = (emb.reshape(HG, 8, E) * inr.astype(f32)).reshape(HG, 8, NT, 8,
                                                               128)
        emb_ref[0, cc] = jnp.transpose(vc, (0, 2, 1, 3, 4)).reshape(
            HG, NT, 8192)


def _dense_body(ang_ref, dst_ref, w1_ref, b1_ref, w2_ref, b2_ref, g_ref,
                out_ref):
    m = pl.program_id(1)
    f32 = jnp.float32
    x = jnp.concatenate([ang_ref[0], dst_ref[0]], axis=-1).reshape(RT * N, 56)
    h = jnp.maximum(
        lax.dot_general(w1_ref[...], x, (((0,), (1,)), ((), ())),
                        preferred_element_type=f32) + b1_ref[0][:, None], 0.0)
    y = lax.dot_general(w2_ref[...], h, (((0,), (0,)), ((), ())),
                        preferred_element_type=f32) + b2_ref[0][:, None]
    y = y.reshape(H, RT, N)
    g = g_ref[0]
    gcol = jnp.broadcast_to(g[:, None, None], (H, RT, 1))
    out_ref[0] = jnp.concatenate([gcol, y], axis=2)

    @pl.when(m == 0)
    def _():
        out_ref[0, :, 0:1, :] = jnp.broadcast_to(g[:, None, None],
                                                 (H, 1, NP1))


def _sc_body(dense_hbm, emb_hbm, idx_hbm, out_hbm, idx_v, val_v, spmem,
             vbuf, tbuf, sem):
    c = lax.axis_index("c")
    s = lax.axis_index("s")
    start_c = c * HALF0
    for b in range(B):
        pltpu.sync_copy(idx_hbm.at[(b * 2 + c) * NT + s], idx_v)
        for g in range(HG):
            base = b * (HG * REG) + g * REG
            off = base + start_c + s * CH
            pltpu.sync_copy(dense_hbm.at[pl.ds(off, CH)], vbuf)
            pltpu.sync_copy(vbuf, spmem.at[pl.ds(s * CH, CH)])

            @pl.when((s == NT - 1) & (c == 1))
            def _():
                pltpu.sync_copy(
                    dense_hbm.at[pl.ds(base + HALF0 + NT * CH, TAIL)], tbuf)
                pltpu.sync_copy(tbuf, spmem.at[pl.ds(NT * CH, TAIL)])

            pltpu.sync_copy(
                emb_hbm.at[((b * 2 + c) * HG + g) * NT + s], val_v)
            plsc.subcore_barrier()
            pltpu.sync_copy(val_v, spmem.at[idx_v], add=True)
            plsc.subcore_barrier()
            pltpu.sync_copy(spmem.at[pl.ds(s * CH, CH)], vbuf)
            pltpu.sync_copy(vbuf, out_hbm.at[pl.ds(off, CH)])

            @pl.when((s == NT - 1) & (c == 1))
            def _():
                pltpu.sync_copy(spmem.at[pl.ds(NT * CH, TAIL)], tbuf)
                pltpu.sync_copy(
                    tbuf, out_hbm.at[pl.ds(base + HALF0 + NT * CH, TAIL)])

            plsc.subcore_barrier()


def _edge_call(ef_t, ei, nl2, em_f, struct_tab, plip_pro, plip_lig, plip_inter,
               dist_w1, dist_b1, dist_w2, dist_b2):
    f32 = jnp.float32
    full = lambda shp: pl.BlockSpec(shp, lambda b: (0,) * len(shp))
    return pl.pallas_call(
        _edge_body,
        grid=(B,),
        in_specs=[
            pl.BlockSpec((1, 4, E), lambda b: (b, 0, 0)),
            pl.BlockSpec((1, 2, E), lambda b: (b, 0, 0)),
            pl.BlockSpec((1, 1, 1), lambda b: (b, 0, 0)),
            pl.BlockSpec((1, 1, E), lambda b: (b, 0, 0)),
            full((20, H)), full((15, H)), full((15, H)), full((15, H)),
            full((H, 1)), full((1, H)), full((H, H)), full((1, H)),
        ],
        out_specs=[
            pl.BlockSpec((1, 2, HG, NT, 8192),
                         lambda b: (b, 0, 0, 0, 0)),
            pl.BlockSpec((1, 2, NT, 8192), lambda b: (b, 0, 0, 0)),
        ],
        out_shape=[
            jax.ShapeDtypeStruct((B, 2, HG, NT, 8192), f32),
            jax.ShapeDtypeStruct((B, 2, NT, 8192), jnp.int32),
        ],
    )(ef_t, ei, nl2, em_f, struct_tab, plip_pro, plip_lig, plip_inter,
      dist_w1, dist_b1, dist_w2, dist_b2)


def _dense_call(angle_p, dists_p, w1c, b1c, w2c, b2c, gtvd):
    f32 = jnp.float32
    full = lambda shp: pl.BlockSpec(shp, lambda b, m: (0,) * len(shp))
    return pl.pallas_call(
        _dense_body,
        grid=(B, NRT),
        in_specs=[
            pl.BlockSpec((1, RT, N, 28), lambda b, m: (b, m, 0, 0)),
            pl.BlockSpec((1, RT, N, 28), lambda b, m: (b, m, 0, 0)),
            full((56, 64)), full((1, 64)), full((64, H)), full((1, H)),
            full((1, H)),
        ],
        out_specs=pl.BlockSpec((1, H, RT, NP1), lambda b, m: (b, 0, m, 0)),
        out_shape=jax.ShapeDtypeStruct((B, H, NP1, NP1), f32),
    )(angle_p, dists_p, w1c, b1c, w2c, b2c, gtvd)


@functools.cache
def _sc_scatter_fn():
    return pl.kernel(
        _sc_body,
        out_type=jax.ShapeDtypeStruct((B * HG * REG,), jnp.float32),
        mesh=plsc.VectorSubcoreMesh(
            core_axis_name="c", subcore_axis_name="s", num_cores=2),
        scratch_types=[
            pltpu.VMEM((8192,), jnp.int32),
            pltpu.VMEM((8192,), jnp.float32),
            pltpu.VMEM_SHARED((HALF1,), jnp.float32),
            pltpu.VMEM((CH,), jnp.float32),
            pltpu.VMEM((TAIL,), jnp.float32),
            pltpu.SemaphoreType.DMA,
        ],
    )


def kernel(edge_feat, edge_index, node_feat, num_ligand_atoms, angle, dists,
           edge_mask, struct_tab, plip_pro, plip_lig, plip_inter,
           dist_w1, dist_b1, dist_w2, dist_b2, gtvd,
           ang_w1, ang_b1, ang_w2, ang_b2, md_w1, md_b1, md_w2, md_b2):
    f32 = jnp.float32
    ef_t = jnp.transpose(edge_feat.astype(f32), (0, 2, 1))
    ei = edge_index.astype(jnp.int32)
    nl2 = num_ligand_atoms.astype(jnp.int32).reshape(B, 1, 1)
    em_f = edge_mask.astype(f32).reshape(B, 1, E)

    emb7, idx7 = _edge_call(
        ef_t, ei, nl2, em_f, struct_tab, plip_pro, plip_lig, plip_inter,
        dist_w1, dist_b1.reshape(1, H), dist_w2, dist_b2.reshape(1, H))

    w1c = jnp.zeros((56, 64), f32)
    w1c = w1c.at[:28, :32].set(ang_w1.T).at[28:, 32:].set(md_w1.T)
    b1c = jnp.concatenate([ang_b1, md_b1]).reshape(1, 64)
    w2c = jnp.concatenate([ang_w2.T, md_w2.T], axis=0)
    b2c = (ang_b2 + md_b2).reshape(1, H)

    pad = [(0, 0), (1, 0), (0, 0), (0, 0)]
    dense = _dense_call(jnp.pad(angle, pad), jnp.pad(dists, pad),
                        w1c, b1c, w2c, b2c, gtvd)

    out = _sc_scatter_fn()(dense.reshape(B * HG * REG),
                           emb7.reshape(B * 2 * HG * NT, 8192),
                           idx7.reshape(B * 2 * NT, 8192))
    return out.reshape(B, H, NP1, NP1)
